# Initial kernel scaffold; baseline (speedup 1.0000x reference)
import functools
import jax
import jax.numpy as jnp
from jax import lax
from jax.experimental import pallas as pl
from jax.experimental.pallas import tpu as pltpu
from jax.experimental.pallas import tpu_sc as plsc

N = 10000
E = 320000
DIN = 128
DE = 16
D = 128
H = 4
C = 32
L = 3

NC = 2
NS = 16
NW = NC * NS
CH = 256
SUB = CH // 128

NP = 10240
TROWS = NP // NS
E2 = E + N
EP0 = NW * CH * 40
EP1 = NW * CH * 41
R0 = 40
R1 = 41

_mesh = functools.partial(
    plsc.VectorSubcoreMesh, core_axis_name="c", subcore_axis_name="s",
    num_cores=NC, num_subcores=NS)


def _seg_kernel(rows_hbm, idx_hbm, z128, z16, acc_hbm, cnt_hbm,
                idx_v, row_v, one_v, acc_s, cnt_s):
    cid = lax.axis_index("c")
    sid = lax.axis_index("s")
    wid = cid * NS + sid
    pltpu.sync_copy(z128.at[pl.ds(sid * TROWS, TROWS)],
                    acc_s.at[pl.ds(sid * TROWS, TROWS)])
    pltpu.sync_copy(z16.at[pl.ds(sid * TROWS, TROWS)],
                    cnt_s.at[pl.ds(sid * TROWS, TROWS)])

    def fill(e, _):
        one_v[e, :] = jnp.ones((16,), jnp.float32)
        return 0
    lax.fori_loop(0, CH, fill, 0)
    plsc.subcore_barrier()

    def rnd(r, _):
        row0 = (wid * R0 + r) * SUB
        pltpu.sync_copy(idx_hbm.at[pl.ds(row0, SUB)], idx_v)
        pltpu.sync_copy(rows_hbm.at[pl.ds(row0 * 128, CH)], row_v)
        for j in range(SUB):
            pltpu.sync_copy(row_v.at[pl.ds(j * 128, 128)],
                            acc_s.at[idx_v.at[j]], add=True)
            pltpu.sync_copy(one_v.at[pl.ds(j * 128, 128)],
                            cnt_s.at[idx_v.at[j]], add=True)
        return 0
    lax.fori_loop(0, R0, rnd, 0)

    plsc.subcore_barrier()
    pltpu.sync_copy(acc_s.at[pl.ds(sid * TROWS, TROWS)],
                    acc_hbm.at[cid, pl.ds(sid * TROWS, TROWS)])
    pltpu.sync_copy(cnt_s.at[pl.ds(sid * TROWS, TROWS)],
                    cnt_hbm.at[cid, pl.ds(sid * TROWS, TROWS)])


def _seg_sum(rows, idx2d, z128, z16):
    k = pl.kernel(
        _seg_kernel,
        out_type=(jax.ShapeDtypeStruct((NC, NP, 128), jnp.float32),
                  jax.ShapeDtypeStruct((NC, NP, 16), jnp.float32)),
        mesh=_mesh(),
        scratch_types=[
            pltpu.VMEM((SUB, 128), jnp.int32),
            pltpu.VMEM((CH, 128), jnp.float32),
            pltpu.VMEM((CH, 16), jnp.float32),
            pltpu.VMEM_SHARED((NP, 128), jnp.float32),
            pltpu.VMEM_SHARED((NP, 16), jnp.float32),
        ],
    )
    return k(rows, idx2d, z128, z16)


def _gat_kernel(xl_hbm, xr_hbm, ee_hbm, sidx_hbm, didx_hbm, att_hbm, z128, z16,
                msg_hbm, den_hbm,
                sidx_v, didx_v, xl_v, xr_v, ee_v, den_v, att_v, acc_s, den_s):
    cid = lax.axis_index("c")
    sid = lax.axis_index("s")
    wid = cid * NS + sid
    pltpu.sync_copy(z128.at[pl.ds(sid * TROWS, TROWS)],
                    acc_s.at[pl.ds(sid * TROWS, TROWS)])
    pltpu.sync_copy(z16.at[pl.ds(sid * TROWS, TROWS)],
                    den_s.at[pl.ds(sid * TROWS, TROWS)])
    pltpu.sync_copy(att_hbm, att_v)
    plsc.subcore_barrier()

    lane = lax.iota(jnp.int32, 16)

    def rnd(r, _):
        row0 = (wid * R1 + r) * SUB
        e0 = row0 * 128
        pltpu.sync_copy(sidx_hbm.at[pl.ds(row0, SUB)], sidx_v)
        pltpu.sync_copy(didx_hbm.at[pl.ds(row0, SUB)], didx_v)
        pltpu.sync_copy(ee_hbm.at[pl.ds(e0, CH)], ee_v)
        for j in range(SUB):
            pltpu.sync_copy(xl_hbm.at[sidx_v.at[j]],
                            xl_v.at[pl.ds(j * 128, 128)])
            pltpu.sync_copy(xr_hbm.at[didx_v.at[j]],
                            xr_v.at[pl.ds(j * 128, 128)])

        a = [att_v[k] for k in range(2 * H)]

        def edge(e, _):
            acc_w = jnp.zeros((16,), jnp.float32)
            for h in range(H):
                l0 = xl_v[e, pl.ds(32 * h, 16)]
                l1 = xl_v[e, pl.ds(32 * h + 16, 16)]
                m0 = l0 + xr_v[e, pl.ds(32 * h, 16)] + ee_v[e, pl.ds(32 * h, 16)]
                m1 = l1 + xr_v[e, pl.ds(32 * h + 16, 16)] + ee_v[e, pl.ds(32 * h + 16, 16)]
                m0 = jnp.where(m0 >= 0.0, m0, 0.2 * m0)
                m1 = jnp.where(m1 >= 0.0, m1, 0.2 * m1)
                logit = jnp.sum(m0 * a[2 * h] + m1 * a[2 * h + 1])
                w = jnp.exp(jnp.full((16,), logit, jnp.float32))
                xl_v[e, pl.ds(32 * h, 16)] = l0 * w
                xl_v[e, pl.ds(32 * h + 16, 16)] = l1 * w
                acc_w = acc_w + jnp.where(lane == h, w, 0.0)
            den_v[e, :] = acc_w
            return 0
        lax.fori_loop(0, CH, edge, 0)

        for j in range(SUB):
            pltpu.sync_copy(xl_v.at[pl.ds(j * 128, 128)],
                            acc_s.at[didx_v.at[j]], add=True)
            pltpu.sync_copy(den_v.at[pl.ds(j * 128, 128)],
                            den_s.at[didx_v.at[j]], add=True)
        return 0
    lax.fori_loop(0, R1, rnd, 0)

    plsc.subcore_barrier()
    pltpu.sync_copy(acc_s.at[pl.ds(sid * TROWS, TROWS)],
                    msg_hbm.at[cid, pl.ds(sid * TROWS, TROWS)])
    pltpu.sync_copy(den_s.at[pl.ds(sid * TROWS, TROWS)],
                    den_hbm.at[cid, pl.ds(sid * TROWS, TROWS)])


def _gat_pass(xl, xr, ee, sidx2d, didx2d, attv, z128, z16):
    k = pl.kernel(
        _gat_kernel,
        out_type=(jax.ShapeDtypeStruct((NC, NP, 128), jnp.float32),
                  jax.ShapeDtypeStruct((NC, NP, 16), jnp.float32)),
        mesh=_mesh(),
        scratch_types=[
            pltpu.VMEM((SUB, 128), jnp.int32),
            pltpu.VMEM((SUB, 128), jnp.int32),
            pltpu.VMEM((CH, 128), jnp.float32),
            pltpu.VMEM((CH, 128), jnp.float32),
            pltpu.VMEM((CH, 128), jnp.float32),
            pltpu.VMEM((CH, 16), jnp.float32),
            pltpu.VMEM((2 * H, 16), jnp.float32),
            pltpu.VMEM_SHARED((NP, 128), jnp.float32),
            pltpu.VMEM_SHARED((NP, 16), jnp.float32),
        ],
    )
    return k(xl, xr, ee, sidx2d, didx2d, attv, z128, z16)


def _mm_body(x_ref, w_ref, b_ref, o_ref):
    o_ref[...] = jnp.dot(x_ref[...], w_ref[...],
                         preferred_element_type=jnp.float32) + b_ref[...]


def _matmul_bias(x, w, b, block=512):
    m, kk = x.shape
    dout = w.shape[1]
    return pl.pallas_call(
        _mm_body,
        grid=(m // block,),
        in_specs=[pl.BlockSpec((block, kk), lambda i: (i, 0)),
                  pl.BlockSpec((kk, dout), lambda i: (0, 0)),
                  pl.BlockSpec((1, dout), lambda i: (0, 0))],
        out_specs=pl.BlockSpec((block, dout), lambda i: (i, 0)),
        out_shape=jax.ShapeDtypeStruct((m, dout), jnp.float32),
    )(x, w, b.reshape(1, dout))


def _loopdiv_body(a0, a1, c0, c1, o):
    c = c0[...][:, :1] + c1[...][:, :1]
    o[...] = (a0[...] + a1[...]) / jnp.maximum(c, 1.0)


def _loop_mean(acc, cnt, block=512):
    return pl.pallas_call(
        _loopdiv_body,
        grid=(NP // block,),
        in_specs=[pl.BlockSpec((block, 128), lambda i: (i, 0)),
                  pl.BlockSpec((block, 128), lambda i: (i, 0)),
                  pl.BlockSpec((block, 16), lambda i: (i, 0)),
                  pl.BlockSpec((block, 16), lambda i: (i, 0))],
        out_specs=pl.BlockSpec((block, 128), lambda i: (i, 0)),
        out_shape=jax.ShapeDtypeStruct((NP, 128), jnp.float32),
    )(acc[0], acc[1], cnt[0], cnt[1])


def _combine_body(m0, m1, d0, d1, hin, gb, bm, bv, bg, bb, o, *, resid):
    msg = m0[...] + m1[...]
    den = d0[...] + d1[...]
    parts = []
    for h in range(H):
        dh = jnp.maximum(den[:, h:h + 1], 1e-30)
        parts.append(msg[:, 32 * h:32 * h + 32] / dh)
    out = jnp.concatenate(parts, axis=1) + gb[...]
    hh = (out - bm[...]) / jnp.sqrt(bv[...] + 1e-5) * bg[...] + bb[...]
    hh = jnp.where(hh > 0.0, hh, jnp.exp(jnp.minimum(hh, 0.0)) - 1.0)
    if resid:
        hh = hh + hin[...]
    o[...] = hh


def _combine(msg, den, hin, gb, bm, bv, bg, bb, resid, block=512):
    vec = pl.BlockSpec((1, 128), lambda i: (0, 0))
    return pl.pallas_call(
        functools.partial(_combine_body, resid=resid),
        grid=(NP // block,),
        in_specs=[pl.BlockSpec((block, 128), lambda i: (i, 0)),
                  pl.BlockSpec((block, 128), lambda i: (i, 0)),
                  pl.BlockSpec((block, 16), lambda i: (i, 0)),
                  pl.BlockSpec((block, 16), lambda i: (i, 0)),
                  pl.BlockSpec((block, 128), lambda i: (i, 0)),
                  vec, vec, vec, vec, vec],
        out_specs=pl.BlockSpec((block, 128), lambda i: (i, 0)),
        out_shape=jax.ShapeDtypeStruct((NP, 128), jnp.float32),
    )(msg[0], msg[1], den[0], den[1], hin,
      gb.reshape(1, 128), bm.reshape(1, 128), bv.reshape(1, 128),
      bg.reshape(1, 128), bb.reshape(1, 128))


def kernel(x, edge_index, edge_attr, in_W, in_b, ep_W, ep_b, Wl, bl, Wr, br,
           We, att, gat_b, bn_g, bn_b, bn_m, bn_v, op_W, op_b):
    src, dst = edge_index[0], edge_index[1]
    z128 = jnp.zeros((NP, 128), jnp.float32)
    z16 = jnp.zeros((NP, 16), jnp.float32)
    ar = jnp.arange(N, dtype=jnp.int32)

    xp = jnp.concatenate([x, jnp.zeros((NP - N, DIN), jnp.float32)], axis=0)
    hp = _matmul_bias(xp, in_W, in_b)

    eap = jnp.concatenate(
        [edge_attr, jnp.zeros((EP0 - E, DE), jnp.float32)], axis=0)
    eemb_p = _matmul_bias(eap, ep_W, ep_b)

    dst0 = jnp.concatenate([dst, jnp.full((EP0 - E,), N, jnp.int32)])
    acc, cnt = _seg_sum(eemb_p, dst0.reshape(EP0 // 128, 128), z128, z16)
    loop = _loop_mean(acc, cnt)

    src2 = jnp.concatenate([src, ar, jnp.full((EP1 - E2,), N, jnp.int32)])
    dst2 = jnp.concatenate([dst, ar, jnp.full((EP1 - E2,), N, jnp.int32)])
    sidx2d = src2.reshape(EP1 // 128, 128)
    didx2d = dst2.reshape(EP1 // 128, 128)
    eemb2 = jnp.concatenate(
        [eemb_p[:E], loop[:N],
         jnp.zeros((EP1 - E2, 128), jnp.float32)], axis=0)

    for i in range(L):
        h_in = hp
        xl = _matmul_bias(hp, Wl[i], bl[i])
        xr = _matmul_bias(hp, Wr[i], br[i])
        ee = _matmul_bias(eemb2, We[i], jnp.zeros((128,), jnp.float32))
        attv = att[i].reshape(2 * H, 16)
        msg, den = _gat_pass(xl, xr, ee, sidx2d, didx2d, attv, z128, z16)
        hp = _combine(msg, den, h_in, gat_b[i], bn_m[i], bn_v[i],
                      bn_g[i], bn_b[i], resid=(i + 1) in (2, 3))

    out = _matmul_bias(hp, op_W, op_b)
    return out[:N]


# speedup vs baseline: 8.7427x; 8.7427x over previous
import functools
import jax
import jax.numpy as jnp
from jax import lax
from jax.experimental import pallas as pl
from jax.experimental.pallas import tpu as pltpu
from jax.experimental.pallas import tpu_sc as plsc

N = 10000
E = 320000
DIN = 128
DE = 16
D = 128
H = 4
C = 32
L = 3

NC = 2
NS = 16
NW = NC * NS
CH = 256
SUB = CH // 128

NP = 10240
TROWS = NP // NS
E2 = E + N
IR0 = 80
IR1 = 88
EP0 = NW * IR0 * 128
EP1 = NW * IR1 * 128
R0 = IR0 // SUB
R1 = IR1 // SUB

_mesh = functools.partial(
    plsc.VectorSubcoreMesh, core_axis_name="c", subcore_axis_name="s",
    num_cores=NC, num_subcores=NS)


def _seg_kernel(rows_hbm, idx_hbm, z64, z16, acc_hbm, cnt_hbm,
                idx_v, row_v, one_v, acc_s, cnt_s):
    cid = lax.axis_index("c")
    sid = lax.axis_index("s")
    wid = cid * NS + sid
    pltpu.sync_copy(z64.at[pl.ds(sid * TROWS, TROWS)],
                    acc_s.at[pl.ds(sid * TROWS, TROWS)])
    pltpu.sync_copy(z16.at[pl.ds(sid * TROWS, TROWS)],
                    cnt_s.at[pl.ds(sid * TROWS, TROWS)])
    pltpu.sync_copy(idx_hbm.at[pl.ds(wid * IR0, IR0)], idx_v)

    def fill(e, _):
        one_v[e, :] = jnp.ones((16,), jnp.float32)
        return 0
    lax.fori_loop(0, CH, fill, 0)
    plsc.subcore_barrier()

    def rnd(r, _):
        e0 = (wid * IR0 + r * SUB) * 128
        pltpu.sync_copy(rows_hbm.at[pl.ds(e0, CH)], row_v)
        for j in range(SUB):
            pltpu.sync_copy(row_v.at[pl.ds(j * 128, 128)],
                            acc_s.at[idx_v.at[r * SUB + j]], add=True)
            pltpu.sync_copy(one_v.at[pl.ds(j * 128, 128)],
                            cnt_s.at[idx_v.at[r * SUB + j]], add=True)
        return 0
    lax.fori_loop(0, R0, rnd, 0)

    plsc.subcore_barrier()
    pltpu.sync_copy(acc_s.at[pl.ds(sid * TROWS, TROWS)],
                    acc_hbm.at[cid, pl.ds(sid * TROWS, TROWS)])
    pltpu.sync_copy(cnt_s.at[pl.ds(sid * TROWS, TROWS)],
                    cnt_hbm.at[cid, pl.ds(sid * TROWS, TROWS)])


def _seg_sum(rows, idx2d, z64, z16):
    k = pl.kernel(
        _seg_kernel,
        out_type=(jax.ShapeDtypeStruct((NC, NP, 64), jnp.float32),
                  jax.ShapeDtypeStruct((NC, NP, 16), jnp.float32)),
        mesh=_mesh(),
        compiler_params=pltpu.CompilerParams(use_tc_tiling_on_sc=False),
        scratch_types=[
            pltpu.VMEM((IR0, 128), jnp.int32),
            pltpu.VMEM((CH, 64), jnp.float32),
            pltpu.VMEM((CH, 16), jnp.float32),
            pltpu.VMEM_SHARED((NP, 64), jnp.float32),
            pltpu.VMEM_SHARED((NP, 16), jnp.float32),
        ],
    )
    return k(rows, idx2d, z64, z16)


def _gat_kernel(xl_hbm, xr_hbm, ee_hbm, sidx_hbm, didx_hbm, att_hbm, z64, z16,
                msg_hbm, den_hbm,
                sidx_v, didx_v, xl_v, xr_v, ee_v, den_v, att_v, acc_s, den_s):
    cid = lax.axis_index("c")
    sid = lax.axis_index("s")
    wid = cid * NS + sid
    pltpu.sync_copy(z64.at[pl.ds(sid * TROWS, TROWS)],
                    acc_s.at[pl.ds(sid * TROWS, TROWS)])
    pltpu.sync_copy(z16.at[pl.ds(sid * TROWS, TROWS)],
                    den_s.at[pl.ds(sid * TROWS, TROWS)])
    pltpu.sync_copy(att_hbm, att_v)
    pltpu.sync_copy(sidx_hbm.at[pl.ds(wid * IR1, IR1)], sidx_v)
    pltpu.sync_copy(didx_hbm.at[pl.ds(wid * IR1, IR1)], didx_v)
    plsc.subcore_barrier()

    lane = lax.iota(jnp.int32, 16)

    def rnd(r, _):
        e0 = (wid * IR1 + r * SUB) * 128
        pltpu.sync_copy(ee_hbm.at[pl.ds(e0, CH)], ee_v)
        for j in range(SUB):
            pltpu.sync_copy(xl_hbm.at[sidx_v.at[r * SUB + j]],
                            xl_v.at[pl.ds(j * 128, 128)])
            pltpu.sync_copy(xr_hbm.at[didx_v.at[r * SUB + j]],
                            xr_v.at[pl.ds(j * 128, 128)])

        a = [att_v[k] for k in range(4)]

        def edge(e, _):
            acc_w = jnp.zeros((16,), jnp.float32)
            for h in range(2):
                l0 = xl_v[e, pl.ds(32 * h, 16)]
                l1 = xl_v[e, pl.ds(32 * h + 16, 16)]
                m0 = l0 + xr_v[e, pl.ds(32 * h, 16)] + ee_v[e, pl.ds(32 * h, 16)]
                m1 = l1 + xr_v[e, pl.ds(32 * h + 16, 16)] + ee_v[e, pl.ds(32 * h + 16, 16)]
                m0 = jnp.where(m0 >= 0.0, m0, 0.2 * m0)
                m1 = jnp.where(m1 >= 0.0, m1, 0.2 * m1)
                s = m0 * a[2 * h] + m1 * a[2 * h + 1]
                for sh in (8, 4, 2, 1):
                    perm = jnp.bitwise_xor(lane, sh)
                    s = s + s.at[perm].get(mode="promise_in_bounds")
                w = jnp.exp(s)
                xl_v[e, pl.ds(32 * h, 16)] = l0 * w
                xl_v[e, pl.ds(32 * h + 16, 16)] = l1 * w
                acc_w = acc_w + jnp.where(lane == h, w, 0.0)
            den_v[e, :] = acc_w
            return 0
        lax.fori_loop(0, CH, edge, 0)

        for j in range(SUB):
            pltpu.sync_copy(xl_v.at[pl.ds(j * 128, 128)],
                            acc_s.at[didx_v.at[r * SUB + j]], add=True)
            pltpu.sync_copy(den_v.at[pl.ds(j * 128, 128)],
                            den_s.at[didx_v.at[r * SUB + j]], add=True)
        return 0
    lax.fori_loop(0, R1, rnd, 0)

    plsc.subcore_barrier()
    pltpu.sync_copy(acc_s.at[pl.ds(sid * TROWS, TROWS)],
                    msg_hbm.at[cid, pl.ds(sid * TROWS, TROWS)])
    pltpu.sync_copy(den_s.at[pl.ds(sid * TROWS, TROWS)],
                    den_hbm.at[cid, pl.ds(sid * TROWS, TROWS)])


def _gat_pass(xl, xr, ee, sidx2d, didx2d, attv, z64, z16):
    k = pl.kernel(
        _gat_kernel,
        out_type=(jax.ShapeDtypeStruct((NC, NP, 64), jnp.float32),
                  jax.ShapeDtypeStruct((NC, NP, 16), jnp.float32)),
        mesh=_mesh(),
        compiler_params=pltpu.CompilerParams(use_tc_tiling_on_sc=False),
        scratch_types=[
            pltpu.VMEM((IR1, 128), jnp.int32),
            pltpu.VMEM((IR1, 128), jnp.int32),
            pltpu.VMEM((CH, 64), jnp.float32),
            pltpu.VMEM((CH, 64), jnp.float32),
            pltpu.VMEM((CH, 64), jnp.float32),
            pltpu.VMEM((CH, 16), jnp.float32),
            pltpu.VMEM((4, 16), jnp.float32),
            pltpu.VMEM_SHARED((NP, 64), jnp.float32),
            pltpu.VMEM_SHARED((NP, 16), jnp.float32),
        ],
    )
    return k(xl, xr, ee, sidx2d, didx2d, attv, z64, z16)


def _mm_body(x_ref, w_ref, b_ref, o_ref):
    o_ref[...] = jnp.dot(x_ref[...], w_ref[...],
                         preferred_element_type=jnp.float32) + b_ref[...]


def _matmul_bias(x, w, b, block=512):
    m, kk = x.shape
    dout = w.shape[1]
    return pl.pallas_call(
        _mm_body,
        grid=(m // block,),
        in_specs=[pl.BlockSpec((block, kk), lambda i: (i, 0)),
                  pl.BlockSpec((kk, dout), lambda i: (0, 0)),
                  pl.BlockSpec((1, dout), lambda i: (0, 0))],
        out_specs=pl.BlockSpec((block, dout), lambda i: (i, 0)),
        out_shape=jax.ShapeDtypeStruct((m, dout), jnp.float32),
    )(x, w, b.reshape(1, dout))


def _loopdiv_body(a0, a1, b0, b1, c0, c1, o):
    c = c0[...][:, :1] + c1[...][:, :1]
    cc = jnp.maximum(c, 1.0)
    o[...] = jnp.concatenate(
        [(a0[...] + a1[...]) / cc, (b0[...] + b1[...]) / cc], axis=1)


def _loop_mean(accA, accB, cnt, block=512):
    b64 = pl.BlockSpec((block, 64), lambda i: (i, 0))
    b16 = pl.BlockSpec((block, 16), lambda i: (i, 0))
    return pl.pallas_call(
        _loopdiv_body,
        grid=(NP // block,),
        in_specs=[b64, b64, b64, b64, b16, b16],
        out_specs=pl.BlockSpec((block, 128), lambda i: (i, 0)),
        out_shape=jax.ShapeDtypeStruct((NP, 128), jnp.float32),
    )(accA[0], accA[1], accB[0], accB[1], cnt[0], cnt[1])


def _combine_body(ma0, ma1, mb0, mb1, da0, da1, db0, db1, hin,
                  gb, bm, bv, bg, bb, o, *, resid):
    msgA = ma0[...] + ma1[...]
    msgB = mb0[...] + mb1[...]
    denA = da0[...] + da1[...]
    denB = db0[...] + db1[...]
    parts = []
    for h in range(2):
        dh = jnp.maximum(denA[:, h:h + 1], 1e-30)
        parts.append(msgA[:, 32 * h:32 * h + 32] / dh)
    for h in range(2):
        dh = jnp.maximum(denB[:, h:h + 1], 1e-30)
        parts.append(msgB[:, 32 * h:32 * h + 32] / dh)
    out = jnp.concatenate(parts, axis=1) + gb[...]
    hh = (out - bm[...]) / jnp.sqrt(bv[...] + 1e-5) * bg[...] + bb[...]
    hh = jnp.where(hh > 0.0, hh, jnp.exp(jnp.minimum(hh, 0.0)) - 1.0)
    if resid:
        hh = hh + hin[...]
    o[...] = hh


def _combine(msgA, msgB, denA, denB, hin, gb, bm, bv, bg, bb, resid,
             block=512):
    vec = pl.BlockSpec((1, 128), lambda i: (0, 0))
    b64 = pl.BlockSpec((block, 64), lambda i: (i, 0))
    b16 = pl.BlockSpec((block, 16), lambda i: (i, 0))
    b128 = pl.BlockSpec((block, 128), lambda i: (i, 0))
    return pl.pallas_call(
        functools.partial(_combine_body, resid=resid),
        grid=(NP // block,),
        in_specs=[b64, b64, b64, b64, b16, b16, b16, b16, b128,
                  vec, vec, vec, vec, vec],
        out_specs=b128,
        out_shape=jax.ShapeDtypeStruct((NP, 128), jnp.float32),
    )(msgA[0], msgA[1], msgB[0], msgB[1],
      denA[0], denA[1], denB[0], denB[1], hin,
      gb.reshape(1, 128), bm.reshape(1, 128), bv.reshape(1, 128),
      bg.reshape(1, 128), bb.reshape(1, 128))


def kernel(x, edge_index, edge_attr, in_W, in_b, ep_W, ep_b, Wl, bl, Wr, br,
           We, att, gat_b, bn_g, bn_b, bn_m, bn_v, op_W, op_b):
    src, dst = edge_index[0], edge_index[1]
    z64 = jnp.zeros((NP, 64), jnp.float32)
    z16 = jnp.zeros((NP, 16), jnp.float32)
    ar = jnp.arange(N, dtype=jnp.int32)

    xp = jnp.concatenate([x, jnp.zeros((NP - N, DIN), jnp.float32)], axis=0)
    hp = _matmul_bias(xp, in_W, in_b)

    eap = jnp.concatenate(
        [edge_attr, jnp.zeros((EP0 - E, DE), jnp.float32)], axis=0)
    eemb_p = _matmul_bias(eap, ep_W, ep_b)
    eembA = _matmul_bias(eap, ep_W[:, :64], ep_b[:64])
    eembB = _matmul_bias(eap, ep_W[:, 64:], ep_b[64:])

    dst0 = jnp.concatenate([dst, jnp.full((EP0 - E,), N, jnp.int32)])
    idx0 = dst0.reshape(EP0 // 128, 128)
    accA, cnt = _seg_sum(eembA, idx0, z64, z16)
    accB, _ = _seg_sum(eembB, idx0, z64, z16)
    loop = _loop_mean(accA, accB, cnt)

    src2 = jnp.concatenate([src, ar, jnp.full((EP1 - E2,), N, jnp.int32)])
    dst2 = jnp.concatenate([dst, ar, jnp.full((EP1 - E2,), N, jnp.int32)])
    sidx2d = src2.reshape(EP1 // 128, 128)
    didx2d = dst2.reshape(EP1 // 128, 128)
    eemb2 = jnp.concatenate(
        [eemb_p[:E], loop[:N],
         jnp.zeros((EP1 - E2, 128), jnp.float32)], axis=0)

    for i in range(L):
        h_in = hp
        xlA = _matmul_bias(hp, Wl[i][:, :64], bl[i][:64])
        xlB = _matmul_bias(hp, Wl[i][:, 64:], bl[i][64:])
        xrA = _matmul_bias(hp, Wr[i][:, :64], br[i][:64])
        xrB = _matmul_bias(hp, Wr[i][:, 64:], br[i][64:])
        zb = jnp.zeros((64,), jnp.float32)
        eeA = _matmul_bias(eemb2, We[i][:, :64], zb)
        eeB = _matmul_bias(eemb2, We[i][:, 64:], zb)
        attv = att[i].reshape(2 * H, 16)
        msgA, denA = _gat_pass(xlA, xrA, eeA, sidx2d, didx2d, attv[:4],
                               z64, z16)
        msgB, denB = _gat_pass(xlB, xrB, eeB, sidx2d, didx2d, attv[4:],
                               z64, z16)
        hp = _combine(msgA, msgB, denA, denB, h_in, gat_b[i], bn_m[i],
                      bn_v[i], bn_g[i], bn_b[i], resid=(i + 1) in (2, 3))

    out = _matmul_bias(hp, op_W, op_b)
    return out[:N]


# speedup vs baseline: 10.2850x; 1.1764x over previous
import functools
import jax
import jax.numpy as jnp
from jax import lax
from jax.experimental import pallas as pl
from jax.experimental.pallas import tpu as pltpu
from jax.experimental.pallas import tpu_sc as plsc

N = 10000
E = 320000
DIN = 128
DE = 16
D = 128
H = 4
C = 32
L = 3

NC = 2
NS = 16
NW = NC * NS
CH = 256
SUB = CH // 128

NP = 10240
TROWS = NP // NS
E2 = E + N
IR0 = 80
IR1 = 88
EP0 = NW * IR0 * 128
EP1 = NW * IR1 * 128
R0 = IR0 // SUB
R1 = IR1 // SUB

_mesh = functools.partial(
    plsc.VectorSubcoreMesh, core_axis_name="c", subcore_axis_name="s",
    num_cores=NC, num_subcores=NS)


def _seg_kernel(rows_hbm, idx_hbm, z64, z16, acc_hbm, cnt_hbm,
                idx_v, row_v, one_v, acc_s, cnt_s, sem):
    cid = lax.axis_index("c")
    sid = lax.axis_index("s")
    wid = cid * NS + sid
    pltpu.sync_copy(z64.at[pl.ds(sid * TROWS, TROWS)],
                    acc_s.at[pl.ds(sid * TROWS, TROWS)])
    pltpu.sync_copy(z16.at[pl.ds(sid * TROWS, TROWS)],
                    cnt_s.at[pl.ds(sid * TROWS, TROWS)])
    pltpu.sync_copy(idx_hbm.at[pl.ds(wid * IR0, IR0)], idx_v)

    def fill(e, _):
        one_v[e, :] = jnp.ones((16,), jnp.float32)
        return 0
    lax.fori_loop(0, CH, fill, 0)
    plsc.subcore_barrier()

    def rnd(r, _):
        e0 = (wid * IR0 + r * SUB) * 128
        pltpu.sync_copy(rows_hbm.at[pl.ds(e0, CH)], row_v)
        ss = []
        for j in range(SUB):
            ss.append(pltpu.async_copy(
                row_v.at[pl.ds(j * 128, 128)],
                acc_s.at[idx_v.at[r * SUB + j]], sem, add=True))
            ss.append(pltpu.async_copy(
                one_v.at[pl.ds(j * 128, 128)],
                cnt_s.at[idx_v.at[r * SUB + j]], sem, add=True))
        for s in ss:
            s.wait()
        return 0
    lax.fori_loop(0, R0, rnd, 0)

    plsc.subcore_barrier()
    pltpu.sync_copy(acc_s.at[pl.ds(sid * TROWS, TROWS)],
                    acc_hbm.at[cid, pl.ds(sid * TROWS, TROWS)])
    pltpu.sync_copy(cnt_s.at[pl.ds(sid * TROWS, TROWS)],
                    cnt_hbm.at[cid, pl.ds(sid * TROWS, TROWS)])


def _seg_sum(rows, idx2d, z64, z16):
    k = pl.kernel(
        _seg_kernel,
        out_type=(jax.ShapeDtypeStruct((NC, NP, 64), jnp.float32),
                  jax.ShapeDtypeStruct((NC, NP, 16), jnp.float32)),
        mesh=_mesh(),
        compiler_params=pltpu.CompilerParams(use_tc_tiling_on_sc=False),
        scratch_types=[
            pltpu.VMEM((IR0, 128), jnp.int32),
            pltpu.VMEM((CH, 64), jnp.float32),
            pltpu.VMEM((CH, 16), jnp.float32),
            pltpu.VMEM_SHARED((NP, 64), jnp.float32),
            pltpu.VMEM_SHARED((NP, 16), jnp.float32),
            pltpu.SemaphoreType.DMA,
        ],
    )
    return k(rows, idx2d, z64, z16)


def _gat_kernel(xl_hbm, xr_hbm, ee_hbm, sidx_hbm, didx_hbm, att_hbm, z64, z16,
                msg_hbm, den_hbm,
                sidx_v, didx_v, xl_v, xr_v, ee_v, den_v, att_v, acc_s, den_s, sem):
    cid = lax.axis_index("c")
    sid = lax.axis_index("s")
    wid = cid * NS + sid
    pltpu.sync_copy(z64.at[pl.ds(sid * TROWS, TROWS)],
                    acc_s.at[pl.ds(sid * TROWS, TROWS)])
    pltpu.sync_copy(z16.at[pl.ds(sid * TROWS, TROWS)],
                    den_s.at[pl.ds(sid * TROWS, TROWS)])
    pltpu.sync_copy(att_hbm, att_v)
    pltpu.sync_copy(sidx_hbm.at[pl.ds(wid * IR1, IR1)], sidx_v)
    pltpu.sync_copy(didx_hbm.at[pl.ds(wid * IR1, IR1)], didx_v)
    plsc.subcore_barrier()

    lane = lax.iota(jnp.int32, 16)

    def rnd(r, _):
        e0 = (wid * IR1 + r * SUB) * 128
        gs = [pltpu.async_copy(ee_hbm.at[pl.ds(e0, CH)], ee_v, sem)]
        for j in range(SUB):
            gs.append(pltpu.async_copy(xl_hbm.at[sidx_v.at[r * SUB + j]],
                                       xl_v.at[pl.ds(j * 128, 128)], sem))
            gs.append(pltpu.async_copy(xr_hbm.at[didx_v.at[r * SUB + j]],
                                       xr_v.at[pl.ds(j * 128, 128)], sem))
        for g in gs:
            g.wait()

        a = [att_v[k] for k in range(4)]

        def edge(e, _):
            acc_w = jnp.zeros((16,), jnp.float32)
            for h in range(2):
                l0 = xl_v[e, pl.ds(32 * h, 16)]
                l1 = xl_v[e, pl.ds(32 * h + 16, 16)]
                m0 = l0 + xr_v[e, pl.ds(32 * h, 16)] + ee_v[e, pl.ds(32 * h, 16)]
                m1 = l1 + xr_v[e, pl.ds(32 * h + 16, 16)] + ee_v[e, pl.ds(32 * h + 16, 16)]
                m0 = jnp.where(m0 >= 0.0, m0, 0.2 * m0)
                m1 = jnp.where(m1 >= 0.0, m1, 0.2 * m1)
                s = m0 * a[2 * h] + m1 * a[2 * h + 1]
                for sh in (8, 4, 2, 1):
                    perm = jnp.bitwise_xor(lane, sh)
                    s = s + s.at[perm].get(mode="promise_in_bounds")
                w = jnp.exp(s)
                xl_v[e, pl.ds(32 * h, 16)] = l0 * w
                xl_v[e, pl.ds(32 * h + 16, 16)] = l1 * w
                acc_w = acc_w + jnp.where(lane == h, w, 0.0)
            den_v[e, :] = acc_w
            return 0
        lax.fori_loop(0, CH, edge, 0)

        ss = []
        for j in range(SUB):
            ss.append(pltpu.async_copy(
                xl_v.at[pl.ds(j * 128, 128)],
                acc_s.at[didx_v.at[r * SUB + j]], sem, add=True))
            ss.append(pltpu.async_copy(
                den_v.at[pl.ds(j * 128, 128)],
                den_s.at[didx_v.at[r * SUB + j]], sem, add=True))
        for s in ss:
            s.wait()
        return 0
    lax.fori_loop(0, R1, rnd, 0)

    plsc.subcore_barrier()
    pltpu.sync_copy(acc_s.at[pl.ds(sid * TROWS, TROWS)],
                    msg_hbm.at[cid, pl.ds(sid * TROWS, TROWS)])
    pltpu.sync_copy(den_s.at[pl.ds(sid * TROWS, TROWS)],
                    den_hbm.at[cid, pl.ds(sid * TROWS, TROWS)])


def _gat_pass(xl, xr, ee, sidx2d, didx2d, attv, z64, z16):
    k = pl.kernel(
        _gat_kernel,
        out_type=(jax.ShapeDtypeStruct((NC, NP, 64), jnp.float32),
                  jax.ShapeDtypeStruct((NC, NP, 16), jnp.float32)),
        mesh=_mesh(),
        compiler_params=pltpu.CompilerParams(use_tc_tiling_on_sc=False),
        scratch_types=[
            pltpu.VMEM((IR1, 128), jnp.int32),
            pltpu.VMEM((IR1, 128), jnp.int32),
            pltpu.VMEM((CH, 64), jnp.float32),
            pltpu.VMEM((CH, 64), jnp.float32),
            pltpu.VMEM((CH, 64), jnp.float32),
            pltpu.VMEM((CH, 16), jnp.float32),
            pltpu.VMEM((4, 16), jnp.float32),
            pltpu.VMEM_SHARED((NP, 64), jnp.float32),
            pltpu.VMEM_SHARED((NP, 16), jnp.float32),
            pltpu.SemaphoreType.DMA,
        ],
    )
    return k(xl, xr, ee, sidx2d, didx2d, attv, z64, z16)


def _mm_body(x_ref, w_ref, b_ref, o_ref):
    o_ref[...] = jnp.dot(x_ref[...], w_ref[...],
                         preferred_element_type=jnp.float32) + b_ref[...]


def _matmul_bias(x, w, b, block=512):
    m, kk = x.shape
    dout = w.shape[1]
    return pl.pallas_call(
        _mm_body,
        grid=(m // block,),
        in_specs=[pl.BlockSpec((block, kk), lambda i: (i, 0)),
                  pl.BlockSpec((kk, dout), lambda i: (0, 0)),
                  pl.BlockSpec((1, dout), lambda i: (0, 0))],
        out_specs=pl.BlockSpec((block, dout), lambda i: (i, 0)),
        out_shape=jax.ShapeDtypeStruct((m, dout), jnp.float32),
    )(x, w, b.reshape(1, dout))


def _loopdiv_body(a0, a1, b0, b1, c0, c1, o):
    c = c0[...][:, :1] + c1[...][:, :1]
    cc = jnp.maximum(c, 1.0)
    o[...] = jnp.concatenate(
        [(a0[...] + a1[...]) / cc, (b0[...] + b1[...]) / cc], axis=1)


def _loop_mean(accA, accB, cnt, block=512):
    b64 = pl.BlockSpec((block, 64), lambda i: (i, 0))
    b16 = pl.BlockSpec((block, 16), lambda i: (i, 0))
    return pl.pallas_call(
        _loopdiv_body,
        grid=(NP // block,),
        in_specs=[b64, b64, b64, b64, b16, b16],
        out_specs=pl.BlockSpec((block, 128), lambda i: (i, 0)),
        out_shape=jax.ShapeDtypeStruct((NP, 128), jnp.float32),
    )(accA[0], accA[1], accB[0], accB[1], cnt[0], cnt[1])


def _combine_body(ma0, ma1, mb0, mb1, da0, da1, db0, db1, hin,
                  gb, bm, bv, bg, bb, o, *, resid):
    msgA = ma0[...] + ma1[...]
    msgB = mb0[...] + mb1[...]
    denA = da0[...] + da1[...]
    denB = db0[...] + db1[...]
    parts = []
    for h in range(2):
        dh = jnp.maximum(denA[:, h:h + 1], 1e-30)
        parts.append(msgA[:, 32 * h:32 * h + 32] / dh)
    for h in range(2):
        dh = jnp.maximum(denB[:, h:h + 1], 1e-30)
        parts.append(msgB[:, 32 * h:32 * h + 32] / dh)
    out = jnp.concatenate(parts, axis=1) + gb[...]
    hh = (out - bm[...]) / jnp.sqrt(bv[...] + 1e-5) * bg[...] + bb[...]
    hh = jnp.where(hh > 0.0, hh, jnp.exp(jnp.minimum(hh, 0.0)) - 1.0)
    if resid:
        hh = hh + hin[...]
    o[...] = hh


def _combine(msgA, msgB, denA, denB, hin, gb, bm, bv, bg, bb, resid,
             block=512):
    vec = pl.BlockSpec((1, 128), lambda i: (0, 0))
    b64 = pl.BlockSpec((block, 64), lambda i: (i, 0))
    b16 = pl.BlockSpec((block, 16), lambda i: (i, 0))
    b128 = pl.BlockSpec((block, 128), lambda i: (i, 0))
    return pl.pallas_call(
        functools.partial(_combine_body, resid=resid),
        grid=(NP // block,),
        in_specs=[b64, b64, b64, b64, b16, b16, b16, b16, b128,
                  vec, vec, vec, vec, vec],
        out_specs=b128,
        out_shape=jax.ShapeDtypeStruct((NP, 128), jnp.float32),
    )(msgA[0], msgA[1], msgB[0], msgB[1],
      denA[0], denA[1], denB[0], denB[1], hin,
      gb.reshape(1, 128), bm.reshape(1, 128), bv.reshape(1, 128),
      bg.reshape(1, 128), bb.reshape(1, 128))


def kernel(x, edge_index, edge_attr, in_W, in_b, ep_W, ep_b, Wl, bl, Wr, br,
           We, att, gat_b, bn_g, bn_b, bn_m, bn_v, op_W, op_b):
    src, dst = edge_index[0], edge_index[1]
    z64 = jnp.zeros((NP, 64), jnp.float32)
    z16 = jnp.zeros((NP, 16), jnp.float32)
    ar = jnp.arange(N, dtype=jnp.int32)

    xp = jnp.concatenate([x, jnp.zeros((NP - N, DIN), jnp.float32)], axis=0)
    hp = _matmul_bias(xp, in_W, in_b)

    eap = jnp.concatenate(
        [edge_attr, jnp.zeros((EP0 - E, DE), jnp.float32)], axis=0)
    eemb_p = _matmul_bias(eap, ep_W, ep_b)
    eembA = _matmul_bias(eap, ep_W[:, :64], ep_b[:64])
    eembB = _matmul_bias(eap, ep_W[:, 64:], ep_b[64:])

    dst0 = jnp.concatenate([dst, jnp.full((EP0 - E,), N, jnp.int32)])
    idx0 = dst0.reshape(EP0 // 128, 128)
    accA, cnt = _seg_sum(eembA, idx0, z64, z16)
    accB, _ = _seg_sum(eembB, idx0, z64, z16)
    loop = _loop_mean(accA, accB, cnt)

    src2 = jnp.concatenate([src, ar, jnp.full((EP1 - E2,), N, jnp.int32)])
    dst2 = jnp.concatenate([dst, ar, jnp.full((EP1 - E2,), N, jnp.int32)])
    sidx2d = src2.reshape(EP1 // 128, 128)
    didx2d = dst2.reshape(EP1 // 128, 128)
    eemb2 = jnp.concatenate(
        [eemb_p[:E], loop[:N],
         jnp.zeros((EP1 - E2, 128), jnp.float32)], axis=0)

    for i in range(L):
        h_in = hp
        xlA = _matmul_bias(hp, Wl[i][:, :64], bl[i][:64])
        xlB = _matmul_bias(hp, Wl[i][:, 64:], bl[i][64:])
        xrA = _matmul_bias(hp, Wr[i][:, :64], br[i][:64])
        xrB = _matmul_bias(hp, Wr[i][:, 64:], br[i][64:])
        zb = jnp.zeros((64,), jnp.float32)
        eeA = _matmul_bias(eemb2, We[i][:, :64], zb)
        eeB = _matmul_bias(eemb2, We[i][:, 64:], zb)
        attv = att[i].reshape(2 * H, 16)
        msgA, denA = _gat_pass(xlA, xrA, eeA, sidx2d, didx2d, attv[:4],
                               z64, z16)
        msgB, denB = _gat_pass(xlB, xrB, eeB, sidx2d, didx2d, attv[4:],
                               z64, z16)
        hp = _combine(msgA, msgB, denA, denB, h_in, gat_b[i], bn_m[i],
                      bn_v[i], bn_g[i], bn_b[i], resid=(i + 1) in (2, 3))

    out = _matmul_bias(hp, op_W, op_b)
    return out[:N]


# speedup vs baseline: 10.3896x; 1.0102x over previous
import functools
import jax
import jax.numpy as jnp
from jax import lax
from jax.experimental import pallas as pl
from jax.experimental.pallas import tpu as pltpu
from jax.experimental.pallas import tpu_sc as plsc

N = 10000
E = 320000
DIN = 128
DE = 16
D = 128
H = 4
C = 32
L = 3

NC = 2
NS = 16
NW = NC * NS
CH = 256
SUB = CH // 128

NP = 10240
TROWS = NP // NS
E2 = E + N
IR0 = 80
IR1 = 88
EP0 = NW * IR0 * 128
EP1 = NW * IR1 * 128
R0 = IR0 // SUB
R1 = IR1 // SUB

_mesh = functools.partial(
    plsc.VectorSubcoreMesh, core_axis_name="c", subcore_axis_name="s",
    num_cores=NC, num_subcores=NS)


def _seg_kernel(rows_hbm, idx_hbm, z64, z16, acc_hbm, cnt_hbm,
                idx_v, row_v, one_v, acc_s, cnt_s, sem):
    cid = lax.axis_index("c")
    sid = lax.axis_index("s")
    wid = cid * NS + sid
    pltpu.sync_copy(z64.at[pl.ds(sid * TROWS, TROWS)],
                    acc_s.at[pl.ds(sid * TROWS, TROWS)])
    pltpu.sync_copy(z16.at[pl.ds(sid * TROWS, TROWS)],
                    cnt_s.at[pl.ds(sid * TROWS, TROWS)])
    pltpu.sync_copy(idx_hbm.at[pl.ds(wid * IR0, IR0)], idx_v)

    def fill(e, _):
        one_v[e, :] = jnp.ones((16,), jnp.float32)
        return 0
    lax.fori_loop(0, CH, fill, 0)
    plsc.subcore_barrier()

    def rnd(r, _):
        e0 = (wid * IR0 + r * SUB) * 128
        pltpu.sync_copy(rows_hbm.at[pl.ds(e0, CH)], row_v)
        ss = []
        for j in range(SUB):
            ss.append(pltpu.async_copy(
                row_v.at[pl.ds(j * 128, 128)],
                acc_s.at[idx_v.at[r * SUB + j]], sem, add=True))
            ss.append(pltpu.async_copy(
                one_v.at[pl.ds(j * 128, 128)],
                cnt_s.at[idx_v.at[r * SUB + j]], sem, add=True))
        for s in ss:
            s.wait()
        return 0
    lax.fori_loop(0, R0, rnd, 0)

    plsc.subcore_barrier()
    pltpu.sync_copy(acc_s.at[pl.ds(sid * TROWS, TROWS)],
                    acc_hbm.at[cid, pl.ds(sid * TROWS, TROWS)])
    pltpu.sync_copy(cnt_s.at[pl.ds(sid * TROWS, TROWS)],
                    cnt_hbm.at[cid, pl.ds(sid * TROWS, TROWS)])


def _seg_sum(rows, idx2d, z64, z16):
    k = pl.kernel(
        _seg_kernel,
        out_type=(jax.ShapeDtypeStruct((NC, NP, 64), jnp.float32),
                  jax.ShapeDtypeStruct((NC, NP, 16), jnp.float32)),
        mesh=_mesh(),
        compiler_params=pltpu.CompilerParams(use_tc_tiling_on_sc=False),
        scratch_types=[
            pltpu.VMEM((IR0, 128), jnp.int32),
            pltpu.VMEM((CH, 64), jnp.float32),
            pltpu.VMEM((CH, 16), jnp.float32),
            pltpu.VMEM_SHARED((NP, 64), jnp.float32),
            pltpu.VMEM_SHARED((NP, 16), jnp.float32),
            pltpu.SemaphoreType.DMA,
        ],
    )
    return k(rows, idx2d, z64, z16)


def _gat_kernel(xl_hbm, xr_hbm, ee_hbm, sidx_hbm, didx_hbm, att_hbm, z64, z16,
                msg_hbm, den_hbm,
                sidx_v, didx_v, xl_v, xr_v, ee_v, den_v, att_v, acc_s, den_s, sem):
    cid = lax.axis_index("c")
    sid = lax.axis_index("s")
    wid = cid * NS + sid
    pltpu.sync_copy(z64.at[pl.ds(sid * TROWS, TROWS)],
                    acc_s.at[pl.ds(sid * TROWS, TROWS)])
    pltpu.sync_copy(z16.at[pl.ds(sid * TROWS, TROWS)],
                    den_s.at[pl.ds(sid * TROWS, TROWS)])
    pltpu.sync_copy(att_hbm, att_v)
    pltpu.sync_copy(sidx_hbm.at[pl.ds(wid * IR1, IR1)], sidx_v)
    pltpu.sync_copy(didx_hbm.at[pl.ds(wid * IR1, IR1)], didx_v)
    plsc.subcore_barrier()

    lane = lax.iota(jnp.int32, 16)

    def rnd(r, _):
        e0 = (wid * IR1 + r * SUB) * 128
        gs = [pltpu.async_copy(ee_hbm.at[pl.ds(e0, CH)], ee_v, sem)]
        for j in range(SUB):
            gs.append(pltpu.async_copy(xl_hbm.at[sidx_v.at[r * SUB + j]],
                                       xl_v.at[pl.ds(j * 128, 128)], sem))
            gs.append(pltpu.async_copy(xr_hbm.at[didx_v.at[r * SUB + j]],
                                       xr_v.at[pl.ds(j * 128, 128)], sem))
        for g in gs:
            g.wait()

        a = [att_v[k] for k in range(4)]

        @plsc.parallel_loop(0, CH, 1, unroll=4)
        def _(e):
            acc_w = jnp.zeros((16,), jnp.float32)
            for h in range(2):
                l0 = xl_v[e, pl.ds(32 * h, 16)]
                l1 = xl_v[e, pl.ds(32 * h + 16, 16)]
                m0 = l0 + xr_v[e, pl.ds(32 * h, 16)] + ee_v[e, pl.ds(32 * h, 16)]
                m1 = l1 + xr_v[e, pl.ds(32 * h + 16, 16)] + ee_v[e, pl.ds(32 * h + 16, 16)]
                m0 = jnp.where(m0 >= 0.0, m0, 0.2 * m0)
                m1 = jnp.where(m1 >= 0.0, m1, 0.2 * m1)
                s = m0 * a[2 * h] + m1 * a[2 * h + 1]
                for sh in (8, 4, 2, 1):
                    perm = jnp.bitwise_xor(lane, sh)
                    s = s + s.at[perm].get(mode="promise_in_bounds")
                w = jnp.exp(s)
                xl_v[e, pl.ds(32 * h, 16)] = l0 * w
                xl_v[e, pl.ds(32 * h + 16, 16)] = l1 * w
                acc_w = acc_w + jnp.where(lane == h, w, 0.0)
            den_v[e, :] = acc_w

        ss = []
        for j in range(SUB):
            ss.append(pltpu.async_copy(
                xl_v.at[pl.ds(j * 128, 128)],
                acc_s.at[didx_v.at[r * SUB + j]], sem, add=True))
            ss.append(pltpu.async_copy(
                den_v.at[pl.ds(j * 128, 128)],
                den_s.at[didx_v.at[r * SUB + j]], sem, add=True))
        for s in ss:
            s.wait()
        return 0
    lax.fori_loop(0, R1, rnd, 0)

    plsc.subcore_barrier()
    pltpu.sync_copy(acc_s.at[pl.ds(sid * TROWS, TROWS)],
                    msg_hbm.at[cid, pl.ds(sid * TROWS, TROWS)])
    pltpu.sync_copy(den_s.at[pl.ds(sid * TROWS, TROWS)],
                    den_hbm.at[cid, pl.ds(sid * TROWS, TROWS)])


def _gat_pass(xl, xr, ee, sidx2d, didx2d, attv, z64, z16):
    k = pl.kernel(
        _gat_kernel,
        out_type=(jax.ShapeDtypeStruct((NC, NP, 64), jnp.float32),
                  jax.ShapeDtypeStruct((NC, NP, 16), jnp.float32)),
        mesh=_mesh(),
        compiler_params=pltpu.CompilerParams(use_tc_tiling_on_sc=False),
        scratch_types=[
            pltpu.VMEM((IR1, 128), jnp.int32),
            pltpu.VMEM((IR1, 128), jnp.int32),
            pltpu.VMEM((CH, 64), jnp.float32),
            pltpu.VMEM((CH, 64), jnp.float32),
            pltpu.VMEM((CH, 64), jnp.float32),
            pltpu.VMEM((CH, 16), jnp.float32),
            pltpu.VMEM((4, 16), jnp.float32),
            pltpu.VMEM_SHARED((NP, 64), jnp.float32),
            pltpu.VMEM_SHARED((NP, 16), jnp.float32),
            pltpu.SemaphoreType.DMA,
        ],
    )
    return k(xl, xr, ee, sidx2d, didx2d, attv, z64, z16)


def _mm_body(x_ref, w_ref, b_ref, o_ref):
    o_ref[...] = jnp.dot(x_ref[...], w_ref[...],
                         preferred_element_type=jnp.float32) + b_ref[...]


def _matmul_bias(x, w, b, block=512):
    m, kk = x.shape
    dout = w.shape[1]
    return pl.pallas_call(
        _mm_body,
        grid=(m // block,),
        in_specs=[pl.BlockSpec((block, kk), lambda i: (i, 0)),
                  pl.BlockSpec((kk, dout), lambda i: (0, 0)),
                  pl.BlockSpec((1, dout), lambda i: (0, 0))],
        out_specs=pl.BlockSpec((block, dout), lambda i: (i, 0)),
        out_shape=jax.ShapeDtypeStruct((m, dout), jnp.float32),
    )(x, w, b.reshape(1, dout))


def _loopdiv_body(a0, a1, b0, b1, c0, c1, o):
    c = c0[...][:, :1] + c1[...][:, :1]
    cc = jnp.maximum(c, 1.0)
    o[...] = jnp.concatenate(
        [(a0[...] + a1[...]) / cc, (b0[...] + b1[...]) / cc], axis=1)


def _loop_mean(accA, accB, cnt, block=512):
    b64 = pl.BlockSpec((block, 64), lambda i: (i, 0))
    b16 = pl.BlockSpec((block, 16), lambda i: (i, 0))
    return pl.pallas_call(
        _loopdiv_body,
        grid=(NP // block,),
        in_specs=[b64, b64, b64, b64, b16, b16],
        out_specs=pl.BlockSpec((block, 128), lambda i: (i, 0)),
        out_shape=jax.ShapeDtypeStruct((NP, 128), jnp.float32),
    )(accA[0], accA[1], accB[0], accB[1], cnt[0], cnt[1])


def _combine_body(ma0, ma1, mb0, mb1, da0, da1, db0, db1, hin,
                  gb, bm, bv, bg, bb, o, *, resid):
    msgA = ma0[...] + ma1[...]
    msgB = mb0[...] + mb1[...]
    denA = da0[...] + da1[...]
    denB = db0[...] + db1[...]
    parts = []
    for h in range(2):
        dh = jnp.maximum(denA[:, h:h + 1], 1e-30)
        parts.append(msgA[:, 32 * h:32 * h + 32] / dh)
    for h in range(2):
        dh = jnp.maximum(denB[:, h:h + 1], 1e-30)
        parts.append(msgB[:, 32 * h:32 * h + 32] / dh)
    out = jnp.concatenate(parts, axis=1) + gb[...]
    hh = (out - bm[...]) / jnp.sqrt(bv[...] + 1e-5) * bg[...] + bb[...]
    hh = jnp.where(hh > 0.0, hh, jnp.exp(jnp.minimum(hh, 0.0)) - 1.0)
    if resid:
        hh = hh + hin[...]
    o[...] = hh


def _combine(msgA, msgB, denA, denB, hin, gb, bm, bv, bg, bb, resid,
             block=512):
    vec = pl.BlockSpec((1, 128), lambda i: (0, 0))
    b64 = pl.BlockSpec((block, 64), lambda i: (i, 0))
    b16 = pl.BlockSpec((block, 16), lambda i: (i, 0))
    b128 = pl.BlockSpec((block, 128), lambda i: (i, 0))
    return pl.pallas_call(
        functools.partial(_combine_body, resid=resid),
        grid=(NP // block,),
        in_specs=[b64, b64, b64, b64, b16, b16, b16, b16, b128,
                  vec, vec, vec, vec, vec],
        out_specs=b128,
        out_shape=jax.ShapeDtypeStruct((NP, 128), jnp.float32),
    )(msgA[0], msgA[1], msgB[0], msgB[1],
      denA[0], denA[1], denB[0], denB[1], hin,
      gb.reshape(1, 128), bm.reshape(1, 128), bv.reshape(1, 128),
      bg.reshape(1, 128), bb.reshape(1, 128))


def kernel(x, edge_index, edge_attr, in_W, in_b, ep_W, ep_b, Wl, bl, Wr, br,
           We, att, gat_b, bn_g, bn_b, bn_m, bn_v, op_W, op_b):
    src, dst = edge_index[0], edge_index[1]
    z64 = jnp.zeros((NP, 64), jnp.float32)
    z16 = jnp.zeros((NP, 16), jnp.float32)
    ar = jnp.arange(N, dtype=jnp.int32)

    xp = jnp.concatenate([x, jnp.zeros((NP - N, DIN), jnp.float32)], axis=0)
    hp = _matmul_bias(xp, in_W, in_b)

    eap = jnp.concatenate(
        [edge_attr, jnp.zeros((EP0 - E, DE), jnp.float32)], axis=0)
    eemb_p = _matmul_bias(eap, ep_W, ep_b)
    eembA = _matmul_bias(eap, ep_W[:, :64], ep_b[:64])
    eembB = _matmul_bias(eap, ep_W[:, 64:], ep_b[64:])

    dst0 = jnp.concatenate([dst, jnp.full((EP0 - E,), N, jnp.int32)])
    idx0 = dst0.reshape(EP0 // 128, 128)
    accA, cnt = _seg_sum(eembA, idx0, z64, z16)
    accB, _ = _seg_sum(eembB, idx0, z64, z16)
    loop = _loop_mean(accA, accB, cnt)

    src2 = jnp.concatenate([src, ar, jnp.full((EP1 - E2,), N, jnp.int32)])
    dst2 = jnp.concatenate([dst, ar, jnp.full((EP1 - E2,), N, jnp.int32)])
    sidx2d = src2.reshape(EP1 // 128, 128)
    didx2d = dst2.reshape(EP1 // 128, 128)
    eemb2 = jnp.concatenate(
        [eemb_p[:E], loop[:N],
         jnp.zeros((EP1 - E2, 128), jnp.float32)], axis=0)

    for i in range(L):
        h_in = hp
        xlA = _matmul_bias(hp, Wl[i][:, :64], bl[i][:64])
        xlB = _matmul_bias(hp, Wl[i][:, 64:], bl[i][64:])
        xrA = _matmul_bias(hp, Wr[i][:, :64], br[i][:64])
        xrB = _matmul_bias(hp, Wr[i][:, 64:], br[i][64:])
        zb = jnp.zeros((64,), jnp.float32)
        eeA = _matmul_bias(eemb2, We[i][:, :64], zb)
        eeB = _matmul_bias(eemb2, We[i][:, 64:], zb)
        attv = att[i].reshape(2 * H, 16)
        msgA, denA = _gat_pass(xlA, xrA, eeA, sidx2d, didx2d, attv[:4],
                               z64, z16)
        msgB, denB = _gat_pass(xlB, xrB, eeB, sidx2d, didx2d, attv[4:],
                               z64, z16)
        hp = _combine(msgA, msgB, denA, denB, h_in, gat_b[i], bn_m[i],
                      bn_v[i], bn_g[i], bn_b[i], resid=(i + 1) in (2, 3))

    out = _matmul_bias(hp, op_W, op_b)
    return out[:N]
